# Initial kernel scaffold; baseline (speedup 1.0000x reference)
import jax
import jax.numpy as jnp
from jax.experimental import pallas as pl


def kernel(node_attr, edge_index, slices, mlp_w1, mlp_b1, mlp_w2, mlp_b2, gru_wi, gru_wh, gru_bi, gru_bh, out_w, out_b):
    raise NotImplementedError("write your pallas kernel here")


# speedup vs baseline: 4.2304x; 4.2304x over previous
import jax, jax.numpy as jnp
from jax.experimental import pallas as pl

N = 10000
DEG = 16
C = 128
T = 3
H = 128
COUT = 128
EPS = 0.01


def _step_body(g_ref, base_ref, cand_ref, noise_ref, b1_ref, w2_ref, b2_ref,
               cur_out_ref):
    base = base_ref[...]
    b1 = b1_ref[...]
    w2b = w2_ref[...].astype(jnp.bfloat16)
    b2 = b2_ref[0, 0]
    cols = []
    for d in range(DEG):
        gd = g_ref[:, d * 64:(d + 1) * 64]
        pre = (base + gd) + b1
        hid = jnp.maximum(pre, 0.0).astype(jnp.bfloat16)
        col = jax.lax.dot_general(hid, w2b, (((1,), (0,)), ((), ())),
                                  preferred_element_type=jnp.float32)
        cols.append(col + b2)
    logp = jnp.concatenate(cols, axis=1)
    amax = jnp.max(logp, axis=1, keepdims=True)
    amax = jnp.where(jnp.isfinite(amax), amax, 0.0)
    norm = jnp.log(jnp.sum(jnp.exp(logp - amax), axis=1, keepdims=True)) + amax
    p = jnp.exp(logp - norm)
    p = p + noise_ref[...]
    m = jnp.max(p, axis=1, keepdims=True)
    iota = jax.lax.broadcasted_iota(jnp.int32, p.shape, 1)
    idx = jnp.min(jnp.where(p >= m, iota, DEG), axis=1, keepdims=True)
    onehot = iota == idx
    cur = jnp.sum(jnp.where(onehot, cand_ref[...], 0), axis=1, keepdims=True)
    cur_out_ref[...] = jnp.broadcast_to(cur, cur_out_ref.shape)


def _select_step(g2, base, cand, noise, b1, w2, b2, block=1000, interpret=False):
    n = g2.shape[0]
    grid = n // block
    cur8 = pl.pallas_call(
        _step_body,
        grid=(grid,),
        in_specs=[
            pl.BlockSpec((block, DEG * 64), lambda i: (i, 0)),
            pl.BlockSpec((block, 64), lambda i: (i, 0)),
            pl.BlockSpec((block, DEG), lambda i: (i, 0)),
            pl.BlockSpec((block, DEG), lambda i: (i, 0)),
            pl.BlockSpec((1, 64), lambda i: (0, 0)),
            pl.BlockSpec((64, 1), lambda i: (0, 0)),
            pl.BlockSpec((1, 1), lambda i: (0, 0)),
        ],
        out_specs=pl.BlockSpec((block, 8), lambda i: (i, 0)),
        out_shape=jax.ShapeDtypeStruct((n, 8), jnp.int32),
        interpret=interpret,
    )(g2, base, cand, noise, b1, w2, b2)
    return cur8[:, 0]


def kernel(node_attr, edge_index, slices, mlp_w1, mlp_b1, mlp_w2, mlp_b2,
           gru_wi, gru_wh, gru_bi, gru_bh, out_w, out_b):
    n, c = node_attr.shape
    starts = slices[:, 0]
    dst = edge_index[1]

    def bdot(a, b):
        return jnp.dot(a.astype(jnp.bfloat16), b.astype(jnp.bfloat16),
                       preferred_element_type=jnp.float32)

    W = mlp_w1.reshape(1 + T, c, 64)
    base = bdot(node_attr, W[0])
    nkey = jax.random.key(1234)
    b1r = mlp_b1.reshape(1, 64)
    w2r = mlp_w2.reshape(64, 1)
    b2r = mlp_b2.reshape(1, 1)
    cur = jnp.arange(n)
    walk_nodes = []
    for ts in range(T):
        p_s = bdot(node_attr, W[1 + ts])
        cand = dst[starts[cur][:, None] + jnp.arange(DEG)[None, :]]
        g2 = p_s[cand].reshape(n, DEG * 64)
        noise = EPS * jax.random.normal(jax.random.fold_in(nkey, ts), (n, DEG),
                                        dtype=jnp.float32)
        cur = _select_step(g2, base, cand.astype(jnp.int32), noise,
                           b1r, w2r, b2r)
        base = base + p_s[cur]
        walk_nodes.append(cur)
    h = jnp.zeros((n, H), dtype=node_attr.dtype)
    embeds = [node_attr] + [jnp.take(node_attr, w, axis=0) for w in walk_nodes]
    for x in embeds:
        gi = x @ gru_wi + gru_bi
        gh = h @ gru_wh + gru_bh
        ir, iz, inn = jnp.split(gi, 3, axis=1)
        hr, hz, hn = jnp.split(gh, 3, axis=1)
        r = jax.nn.sigmoid(ir + hr)
        z = jax.nn.sigmoid(iz + hz)
        ncand = jnp.tanh(inn + r * hn)
        h = (1.0 - z) * ncand + z * h
    return h @ out_w + out_b


# speedup vs baseline: 9.4554x; 2.2351x over previous
import functools
import jax, jax.numpy as jnp
from jax import lax
from jax.experimental import pallas as pl
from jax.experimental.pallas import tpu as pltpu, tpu_sc as plsc

N = 10000
DEG = 16
C = 128
T = 3
H = 128
COUT = 128
EPS = 0.01

NC = 2
NS = 16
NW = NC * NS
NPAD = 10240
NODES_PW = NPAD // NW
CH = 32
NCHUNK = NODES_PW // CH


def _sc_gather_body(cur_hbm, dst2_hbm, proj_hbm, cand_hbm, g_hbm,
                    cur_v, cand_v, candf_v, g_v, sem1, sem2):
    wid = lax.axis_index("s") * NC + lax.axis_index("c")
    row0 = wid * NODES_PW

    @pl.loop(0, NCHUNK)
    def _chunk(k):
        r = row0 + k * CH
        pltpu.sync_copy(cur_hbm.at[pl.ds(r, CH)], cur_v)
        pltpu.async_copy(dst2_hbm.at[cur_v], cand_v, sem1).wait()
        pltpu.sync_copy(cand_v, cand_hbm.at[pl.ds(r, CH)])
        for i in range(CH):
            candf_v[pl.ds(i * DEG, DEG)] = cand_v[i, pl.ds(0, DEG)]
        pltpu.async_copy(proj_hbm.at[candf_v], g_v, sem2).wait()
        pltpu.sync_copy(g_v, g_hbm.at[pl.ds(r * DEG, CH * DEG)])


@functools.partial(
    pl.kernel,
    out_type=[
        jax.ShapeDtypeStruct((NPAD, 128), jnp.int32),
        jax.ShapeDtypeStruct((NPAD * DEG, 128), jnp.float32),
    ],
    mesh=plsc.VectorSubcoreMesh(core_axis_name="c", subcore_axis_name="s"),
    scratch_types=[
        pltpu.VMEM((CH,), jnp.int32),
        pltpu.VMEM((CH, 128), jnp.int32),
        pltpu.VMEM((CH * DEG,), jnp.int32),
        pltpu.VMEM((CH * DEG, 128), jnp.float32),
        pltpu.SemaphoreType.DMA,
        pltpu.SemaphoreType.DMA,
    ],
)
def _sc_gather(cur_hbm, dst2_hbm, proj_hbm, cand_hbm, g_hbm,
               cur_v, cand_v, candf_v, g_v, sem1, sem2):
    _sc_gather_body(cur_hbm, dst2_hbm, proj_hbm, cand_hbm, g_hbm,
                    cur_v, cand_v, candf_v, g_v, sem1, sem2)


def _step_body(g_ref, base_ref, cand_ref, noise_ref, w2_ref, b2_ref, b1_ref,
               cur_out_ref):
    base = base_ref[...]
    b1 = b1_ref[...]
    w2b = w2_ref[...].astype(jnp.bfloat16)
    b2 = b2_ref[0, 0]
    cols = []
    for d in range(DEG):
        gd = g_ref[:, d, 0:64]
        pre = (base + gd) + b1
        hid = jnp.maximum(pre, 0.0).astype(jnp.bfloat16)
        col = jax.lax.dot_general(hid, w2b, (((1,), (0,)), ((), ())),
                                  preferred_element_type=jnp.float32)
        cols.append(col + b2)
    logp = jnp.concatenate(cols, axis=1)
    amax = jnp.max(logp, axis=1, keepdims=True)
    amax = jnp.where(jnp.isfinite(amax), amax, 0.0)
    norm = jnp.log(jnp.sum(jnp.exp(logp - amax), axis=1, keepdims=True)) + amax
    p = jnp.exp(logp - norm)
    p = p + noise_ref[...]
    m = jnp.max(p, axis=1, keepdims=True)
    iota = jax.lax.broadcasted_iota(jnp.int32, p.shape, 1)
    idx = jnp.min(jnp.where(p >= m, iota, DEG), axis=1, keepdims=True)
    onehot = iota == idx
    cand16 = cand_ref[...][:, 0:DEG]
    cur = jnp.sum(jnp.where(onehot, cand16, 0), axis=1, keepdims=True)
    cur_out_ref[...] = jnp.broadcast_to(cur, cur_out_ref.shape)


def _select_step(g, base, cand, noise, w2, b2, b1, block=1024):
    n = g.shape[0]
    grid = n // block
    cur8 = pl.pallas_call(
        _step_body,
        grid=(grid,),
        in_specs=[
            pl.BlockSpec((block, DEG, 128), lambda i: (i, 0, 0)),
            pl.BlockSpec((block, 64), lambda i: (i, 0)),
            pl.BlockSpec((block, 128), lambda i: (i, 0)),
            pl.BlockSpec((block, DEG), lambda i: (i, 0)),
            pl.BlockSpec((64, 1), lambda i: (0, 0)),
            pl.BlockSpec((1, 1), lambda i: (0, 0)),
            pl.BlockSpec((1, 64), lambda i: (0, 0)),
        ],
        out_specs=pl.BlockSpec((block, 8), lambda i: (i, 0)),
        out_shape=jax.ShapeDtypeStruct((n, 8), jnp.int32),
    )(g, base, cand, noise, w2, b2, b1)
    return cur8[:, 0]


def kernel(node_attr, edge_index, slices, mlp_w1, mlp_b1, mlp_w2, mlp_b2,
           gru_wi, gru_wh, gru_bi, gru_bh, out_w, out_b):
    n, c = node_attr.shape
    dst2 = edge_index[1].reshape(n, DEG).astype(jnp.int32)
    dst2p = jnp.pad(dst2, ((0, 0), (0, 128 - DEG)))

    def bdot(a, b):
        return jnp.dot(a.astype(jnp.bfloat16), b.astype(jnp.bfloat16),
                       preferred_element_type=jnp.float32)

    W = mlp_w1.reshape(1 + T, c, 64)
    base0 = bdot(node_attr, W[0])
    base = jnp.concatenate([base0, jnp.zeros((NPAD - n, 64), jnp.float32)], 0)
    nkey = jax.random.key(1234)
    b1r = mlp_b1.reshape(1, 64)
    w2r = mlp_w2.reshape(64, 1)
    b2r = mlp_b2.reshape(1, 1)
    cur = jnp.concatenate([jnp.arange(n, dtype=jnp.int32),
                           jnp.zeros(NPAD - n, jnp.int32)])
    walk_nodes = []
    for ts in range(T):
        p_s = bdot(node_attr, W[1 + ts])
        p_sp = jnp.pad(p_s, ((0, 0), (0, 64)))
        cand, gflat = _sc_gather(cur, dst2p, p_sp)
        g = gflat.reshape(NPAD, DEG, 128)
        noise = EPS * jax.random.normal(jax.random.fold_in(nkey, ts), (n, DEG),
                                        dtype=jnp.float32)
        noise = jnp.concatenate([noise, jnp.zeros((NPAD - n, DEG), jnp.float32)], 0)
        cur = _select_step(g, base, cand, noise, w2r, b2r, b1r)
        base = base + p_s[cur]
        walk_nodes.append(cur[:n])
    h = jnp.zeros((n, H), dtype=node_attr.dtype)
    embeds = [node_attr] + [jnp.take(node_attr, w, axis=0) for w in walk_nodes]
    for x in embeds:
        gi = x @ gru_wi + gru_bi
        gh = h @ gru_wh + gru_bh
        ir, iz, inn = jnp.split(gi, 3, axis=1)
        hr, hz, hn = jnp.split(gh, 3, axis=1)
        r = jax.nn.sigmoid(ir + hr)
        z = jax.nn.sigmoid(iz + hz)
        ncand = jnp.tanh(inn + r * hn)
        h = (1.0 - z) * ncand + z * h
    return h @ out_w + out_b
